# baseline (speedup 1.0000x reference)
import functools

import jax
import jax.numpy as jnp
from jax import lax
from jax.experimental import pallas as pl
from jax.experimental.pallas import tpu as pltpu
from jax.experimental.pallas import tpu_sc as plsc

B, S, D, H = 2, 2048, 4096, 32
DH = D // H
K = 3
EPS = 1e-4
BH = B * H
R = B * H * S
CHUNK = 512
NW = 32
CPW = R // (CHUNK * NW)


def _monotone_key(v):
    b = lax.bitcast_convert_type(v, jnp.int32)
    m = lax.shift_right_arithmetic(b, 31)
    return b ^ (m & jnp.int32(0x7FFFFFFF))


def _hash_body(x_ref, xa_ref, xb2_ref, wha_ref, whb_ref, bha_ref,
               bhb_ref, id_ref, xt_ref, t_ref):
    i = pl.program_id(0)
    e = i % 2
    projA = lax.dot_general(
        xa_ref[0], wha_ref[0], (((1,), (0,)), ((), ())),
        precision=lax.Precision.DEFAULT, preferred_element_type=jnp.float32)
    projB = lax.dot_general(
        xb2_ref[0], whb_ref[0], (((1,), (0,)), ((), ())),
        precision=lax.Precision.DEFAULT, preferred_element_type=jnp.float32)
    hx = jnp.where(e == 0,
                   projA[:, 0:1] + bha_ref[0, 0, 0],
                   projA[:, 1:2] + bha_ref[0, 0, 1])
    hy = jnp.where(e == 0,
                   projB[:, 0:1] + bhb_ref[0, 0, 0],
                   projB[:, 1:2] + bhb_ref[0, 0, 1])
    t_col = hx / (hy + EPS)
    ident = id_ref[...]
    t_row = lax.dot_general(
        t_col, ident, (((0,), (0,)), ((), ())),
        precision=lax.Precision.HIGHEST, preferred_element_type=jnp.float32)
    t_ref[...] = t_row[None]
    xt_ref[...] = x_ref[0]


def _rank_body(a_ref, id_ref, p_ref):
    i = pl.program_id(0)
    ident = id_ref[...]
    kr = _monotone_key(a_ref[0])
    a_col = lax.dot_general(
        ident, a_ref[0], (((1,), (1,)), ((), ())),
        precision=lax.Precision.HIGHEST, preferred_element_type=jnp.float32)
    kc = _monotone_key(a_col)
    ii = lax.broadcasted_iota(jnp.int32, (S, 1), 0)
    acc = jnp.zeros((S, 1), jnp.int32)
    CH = 512
    for jc in range(S // CH):
        kj = kr[:, jc * CH:(jc + 1) * CH]
        jj = lax.broadcasted_iota(jnp.int32, (1, CH), 1) + jc * CH
        hit = (kj < kc) | ((kj == kc) & (jj < ii))
        acc = acc + jnp.sum(hit.astype(jnp.int32), axis=1, keepdims=True)
    p_col = (acc + i * S).astype(jnp.float32)
    p_row = lax.dot_general(
        p_col, ident, (((0,), (0,)), ((), ())),
        precision=lax.Precision.HIGHEST, preferred_element_type=jnp.float32)
    p_ref[...] = p_row.astype(jnp.int32).reshape(S)


def _conv_body(xs_ref, w_ref, bc_ref, y_ref):
    xsb = xs_ref[...]
    dn = jnp.concatenate([xsb[S - 1:], xsb[:S - 1]], axis=0)
    up = jnp.concatenate([xsb[1:], xsb[:1]], axis=0)
    w = w_ref[...]
    dims = (((1,), (1,)), ((), ()))
    y = lax.dot_general(dn, w[:, :, 0], dims,
                        precision=lax.Precision.HIGHEST,
                        preferred_element_type=jnp.float32)
    y += lax.dot_general(xsb, w[:, :, 1], dims,
                         precision=lax.Precision.HIGHEST,
                         preferred_element_type=jnp.float32)
    y += lax.dot_general(up, w[:, :, 2], dims,
                         precision=lax.Precision.HIGHEST,
                         preferred_element_type=jnp.float32)
    y_ref[...] = y + bc_ref[0]


def _sc_permute_body(xt_hbm, p_hbm, xs_hbm, buf, idxv, sem):
    wid = lax.axis_index("s") * 2 + lax.axis_index("c")
    for k in range(CPW):
        start = (wid * CPW + k) * CHUNK
        pltpu.sync_copy(xt_hbm.at[pl.ds(start, CHUNK)], buf)
        pltpu.sync_copy(p_hbm.at[pl.ds(start, CHUNK)], idxv)
        pltpu.async_copy(buf, xs_hbm.at[idxv], sem).wait()


def _sc_unpermute_body(y_hbm, p_hbm, out_hbm, buf, idxv, sem):
    wid = lax.axis_index("s") * 2 + lax.axis_index("c")
    for k in range(CPW):
        chunk = wid * CPW + k
        start = chunk * CHUNK
        bh = chunk // (S // CHUNK)
        s0 = (chunk % (S // CHUNK)) * CHUNK
        b = bh // H
        h = bh % H
        pltpu.sync_copy(p_hbm.at[pl.ds(start, CHUNK)], idxv)
        pltpu.async_copy(y_hbm.at[idxv], buf, sem).wait()
        pltpu.sync_copy(buf, out_hbm.at[pl.ds(b * S + s0, CHUNK), h])


def kernel(x, W_hash, b_hash, W_conv, b_conv):
    ident = jnp.eye(S, dtype=jnp.float32)
    b_hash3 = b_hash.reshape(H, 1, 2)
    b_conv3 = b_conv.reshape(H, 1, DH)

    xt, t3 = pl.pallas_call(
        _hash_body,
        grid=(BH,),
        in_specs=[
            pl.BlockSpec((1, S, DH), lambda i: (i // H, 0, i % H)),
            pl.BlockSpec((1, S, DH), lambda i: (i // H, 0, (i % H) // 2)),
            pl.BlockSpec((1, S, DH),
                         lambda i: (i // H, 0, H // 2 + (i % H) // 2)),
            pl.BlockSpec((1, DH, 2), lambda i: ((i % H) // 2, 0, 0)),
            pl.BlockSpec((1, DH, 2), lambda i: (H // 2 + (i % H) // 2, 0, 0)),
            pl.BlockSpec((1, 1, 2), lambda i: ((i % H) // 2, 0, 0)),
            pl.BlockSpec((1, 1, 2), lambda i: (H // 2 + (i % H) // 2, 0, 0)),
            pl.BlockSpec((S, S), lambda i: (0, 0)),
        ],
        out_specs=[
            pl.BlockSpec((S, DH), lambda i: (i, 0)),
            pl.BlockSpec((1, 1, S), lambda i: (i, 0, 0)),
        ],
        out_shape=[
            jax.ShapeDtypeStruct((R, DH), jnp.float32),
            jax.ShapeDtypeStruct((BH, 1, S), jnp.float32),
        ],
    )(x, x, x, W_hash, W_hash, b_hash3, b_hash3, ident)

    angles3 = jnp.arctan(t3)

    P = pl.pallas_call(
        _rank_body,
        grid=(BH,),
        in_specs=[
            pl.BlockSpec((1, 1, S), lambda i: (i, 0, 0)),
            pl.BlockSpec((S, S), lambda i: (0, 0)),
        ],
        out_specs=pl.BlockSpec((S,), lambda i: (i,)),
        out_shape=jax.ShapeDtypeStruct((R,), jnp.int32),
    )(angles3, ident)

    mesh = plsc.VectorSubcoreMesh(core_axis_name="c", subcore_axis_name="s")

    sc_permute = functools.partial(
        pl.kernel,
        mesh=mesh,
        out_type=jax.ShapeDtypeStruct((R, DH), jnp.float32),
        scratch_types=[
            pltpu.VMEM((CHUNK, DH), jnp.float32),
            pltpu.VMEM((CHUNK,), jnp.int32),
            pltpu.SemaphoreType.DMA,
        ],
    )(_sc_permute_body)
    xs = sc_permute(xt, P)

    y = pl.pallas_call(
        _conv_body,
        grid=(BH,),
        in_specs=[
            pl.BlockSpec((S, DH), lambda i: (i, 0)),
            pl.BlockSpec((DH, DH, K), lambda i: (i % H, 0, 0)),
            pl.BlockSpec((1, 1, DH), lambda i: (i % H, 0, 0)),
        ],
        out_specs=pl.BlockSpec((S, DH), lambda i: (i, 0)),
        out_shape=jax.ShapeDtypeStruct((R, DH), jnp.float32),
    )(xs, W_conv, b_conv3)

    sc_unpermute = functools.partial(
        pl.kernel,
        mesh=mesh,
        out_type=jax.ShapeDtypeStruct((B * S, H, DH), jnp.float32),
        scratch_types=[
            pltpu.VMEM((CHUNK, DH), jnp.float32),
            pltpu.VMEM((CHUNK,), jnp.int32),
            pltpu.SemaphoreType.DMA,
        ],
    )(_sc_unpermute_body)
    out3 = sc_unpermute(y, P)

    return out3.reshape(B, S, D)


# speedup vs baseline: 1.1895x; 1.1895x over previous
import functools

import jax
import jax.numpy as jnp
from jax import lax
from jax.experimental import pallas as pl
from jax.experimental.pallas import tpu as pltpu
from jax.experimental.pallas import tpu_sc as plsc

B, S, D, H = 2, 2048, 4096, 32
DH = D // H
K = 3
EPS = 1e-4
BH = B * H
R = B * H * S
CHUNK = 512
NW = 32
CPW = R // (CHUNK * NW)


def _monotone_key(v):
    b = lax.bitcast_convert_type(v, jnp.int32)
    m = lax.shift_right_arithmetic(b, 31)
    return b ^ (m & jnp.int32(0x7FFFFFFF))


def _hash_body(x_ref, xa_ref, xb2_ref, wha_ref, whb_ref, bha_ref,
               bhb_ref, id_ref, xt_ref, t_ref):
    i = pl.program_id(0)
    e = i % 2
    projA = lax.dot_general(
        xa_ref[0], wha_ref[0], (((1,), (0,)), ((), ())),
        precision=lax.Precision.DEFAULT, preferred_element_type=jnp.float32)
    projB = lax.dot_general(
        xb2_ref[0], whb_ref[0], (((1,), (0,)), ((), ())),
        precision=lax.Precision.DEFAULT, preferred_element_type=jnp.float32)
    hx = jnp.where(e == 0,
                   projA[:, 0:1] + bha_ref[0, 0, 0],
                   projA[:, 1:2] + bha_ref[0, 0, 1])
    hy = jnp.where(e == 0,
                   projB[:, 0:1] + bhb_ref[0, 0, 0],
                   projB[:, 1:2] + bhb_ref[0, 0, 1])
    t_col = hx / (hy + EPS)
    ident = id_ref[...]
    t_row = lax.dot_general(
        t_col, ident, (((0,), (0,)), ((), ())),
        precision=lax.Precision.HIGHEST, preferred_element_type=jnp.float32)
    t_ref[...] = t_row[None]
    xt_ref[...] = x_ref[0]


def _rank_body(a_ref, id_ref, p_ref):
    i = pl.program_id(0)
    ident = id_ref[...]
    kr = _monotone_key(a_ref[0])
    a_col = lax.dot_general(
        ident, a_ref[0], (((1,), (1,)), ((), ())),
        precision=lax.Precision.HIGHEST, preferred_element_type=jnp.float32)
    kc = _monotone_key(a_col)
    ii = lax.broadcasted_iota(jnp.int32, (S, 1), 0)
    acc = jnp.zeros((S, 1), jnp.float32)
    CH = 512
    ones = jnp.ones((CH, 1), jnp.float32)
    for jc in range(S // CH):
        kj = kr[:, jc * CH:(jc + 1) * CH]
        jj = lax.broadcasted_iota(jnp.int32, (1, CH), 1) + jc * CH
        hit = (kj < kc) | ((kj == kc) & (jj < ii))
        acc = acc + lax.dot_general(
            jnp.where(hit, 1.0, 0.0), ones, (((1,), (0,)), ((), ())),
            precision=lax.Precision.DEFAULT,
            preferred_element_type=jnp.float32)
    p_col = acc + jnp.float32(1.0) * (i * S)
    p_row = lax.dot_general(
        p_col, ident, (((0,), (0,)), ((), ())),
        precision=lax.Precision.HIGHEST, preferred_element_type=jnp.float32)
    p_ref[...] = p_row.astype(jnp.int32).reshape(S)


def _conv_body(xs_ref, w_ref, bc_ref, y_ref):
    xsb = xs_ref[...]
    dn = jnp.concatenate([xsb[S - 1:], xsb[:S - 1]], axis=0)
    up = jnp.concatenate([xsb[1:], xsb[:1]], axis=0)
    w = w_ref[...]
    dims = (((1,), (1,)), ((), ()))
    y = lax.dot_general(dn, w[:, :, 0], dims,
                        precision=lax.Precision.DEFAULT,
                        preferred_element_type=jnp.float32)
    y += lax.dot_general(xsb, w[:, :, 1], dims,
                         precision=lax.Precision.DEFAULT,
                         preferred_element_type=jnp.float32)
    y += lax.dot_general(up, w[:, :, 2], dims,
                         precision=lax.Precision.DEFAULT,
                         preferred_element_type=jnp.float32)
    y_ref[...] = y + bc_ref[0]


def _sc_permute_body(xt_hbm, p_hbm, xs_hbm, buf, idxv, sem):
    wid = lax.axis_index("s") * 2 + lax.axis_index("c")
    for k in range(CPW):
        start = (wid * CPW + k) * CHUNK
        pltpu.sync_copy(xt_hbm.at[pl.ds(start, CHUNK)], buf)
        pltpu.sync_copy(p_hbm.at[pl.ds(start, CHUNK)], idxv)
        pltpu.async_copy(buf, xs_hbm.at[idxv], sem).wait()


def _sc_unpermute_body(y_hbm, p_hbm, out_hbm, buf, idxv, sem):
    wid = lax.axis_index("s") * 2 + lax.axis_index("c")
    for k in range(CPW):
        chunk = wid * CPW + k
        start = chunk * CHUNK
        bh = chunk // (S // CHUNK)
        s0 = (chunk % (S // CHUNK)) * CHUNK
        b = bh // H
        h = bh % H
        pltpu.sync_copy(p_hbm.at[pl.ds(start, CHUNK)], idxv)
        pltpu.async_copy(y_hbm.at[idxv], buf, sem).wait()
        pltpu.sync_copy(buf, out_hbm.at[pl.ds(b * S + s0, CHUNK), h])


def kernel(x, W_hash, b_hash, W_conv, b_conv):
    ident = jnp.eye(S, dtype=jnp.float32)
    b_hash3 = b_hash.reshape(H, 1, 2)
    b_conv3 = b_conv.reshape(H, 1, DH)

    xt, t3 = pl.pallas_call(
        _hash_body,
        grid=(BH,),
        in_specs=[
            pl.BlockSpec((1, S, DH), lambda i: (i // H, 0, i % H)),
            pl.BlockSpec((1, S, DH), lambda i: (i // H, 0, (i % H) // 2)),
            pl.BlockSpec((1, S, DH),
                         lambda i: (i // H, 0, H // 2 + (i % H) // 2)),
            pl.BlockSpec((1, DH, 2), lambda i: ((i % H) // 2, 0, 0)),
            pl.BlockSpec((1, DH, 2), lambda i: (H // 2 + (i % H) // 2, 0, 0)),
            pl.BlockSpec((1, 1, 2), lambda i: ((i % H) // 2, 0, 0)),
            pl.BlockSpec((1, 1, 2), lambda i: (H // 2 + (i % H) // 2, 0, 0)),
            pl.BlockSpec((S, S), lambda i: (0, 0)),
        ],
        out_specs=[
            pl.BlockSpec((S, DH), lambda i: (i, 0)),
            pl.BlockSpec((1, 1, S), lambda i: (i, 0, 0)),
        ],
        out_shape=[
            jax.ShapeDtypeStruct((R, DH), jnp.float32),
            jax.ShapeDtypeStruct((BH, 1, S), jnp.float32),
        ],
    )(x, x, x, W_hash, W_hash, b_hash3, b_hash3, ident)

    angles3 = jnp.arctan(t3)

    P = pl.pallas_call(
        _rank_body,
        grid=(BH,),
        in_specs=[
            pl.BlockSpec((1, 1, S), lambda i: (i, 0, 0)),
            pl.BlockSpec((S, S), lambda i: (0, 0)),
        ],
        out_specs=pl.BlockSpec((S,), lambda i: (i,)),
        out_shape=jax.ShapeDtypeStruct((R,), jnp.int32),
    )(angles3, ident)

    mesh = plsc.VectorSubcoreMesh(core_axis_name="c", subcore_axis_name="s")

    sc_permute = functools.partial(
        pl.kernel,
        mesh=mesh,
        out_type=jax.ShapeDtypeStruct((R, DH), jnp.float32),
        scratch_types=[
            pltpu.VMEM((CHUNK, DH), jnp.float32),
            pltpu.VMEM((CHUNK,), jnp.int32),
            pltpu.SemaphoreType.DMA,
        ],
    )(_sc_permute_body)
    xs = sc_permute(xt, P)

    y = pl.pallas_call(
        _conv_body,
        grid=(BH,),
        in_specs=[
            pl.BlockSpec((S, DH), lambda i: (i, 0)),
            pl.BlockSpec((DH, DH, K), lambda i: (i % H, 0, 0)),
            pl.BlockSpec((1, 1, DH), lambda i: (i % H, 0, 0)),
        ],
        out_specs=pl.BlockSpec((S, DH), lambda i: (i, 0)),
        out_shape=jax.ShapeDtypeStruct((R, DH), jnp.float32),
    )(xs, W_conv, b_conv3)

    sc_unpermute = functools.partial(
        pl.kernel,
        mesh=mesh,
        out_type=jax.ShapeDtypeStruct((B * S, H, DH), jnp.float32),
        scratch_types=[
            pltpu.VMEM((CHUNK, DH), jnp.float32),
            pltpu.VMEM((CHUNK,), jnp.int32),
            pltpu.SemaphoreType.DMA,
        ],
    )(_sc_unpermute_body)
    out3 = sc_unpermute(y, P)

    return out3.reshape(B, S, D)


# speedup vs baseline: 2.3678x; 1.9907x over previous
import functools

import jax
import jax.numpy as jnp
from jax import lax
from jax.experimental import pallas as pl
from jax.experimental.pallas import tpu as pltpu
from jax.experimental.pallas import tpu_sc as plsc

B, S, D, H = 2, 2048, 4096, 32
DH = D // H
K = 3
EPS = 1e-4
BH = B * H
R = B * H * S
CHUNK = 512
NW = 32
CPW = R // (CHUNK * NW)


def _monotone_key(v):
    b = lax.bitcast_convert_type(v, jnp.int32)
    m = lax.shift_right_arithmetic(b, 31)
    return b ^ (m & jnp.int32(0x7FFFFFFF))


def _hash_body(x_ref, xa_ref, xb2_ref, wn_ref, wd_ref, bn_ref,
               bd_ref, xt_ref, t_ref):
    dims = (((1,), (1,)), ((), ()))
    hx = lax.dot_general(
        wn_ref[0], xa_ref[0], dims,
        precision=lax.Precision.DEFAULT,
        preferred_element_type=jnp.float32) + bn_ref[0, 0, 0]
    hy = lax.dot_general(
        wd_ref[0], xb2_ref[0], dims,
        precision=lax.Precision.DEFAULT,
        preferred_element_type=jnp.float32) + bd_ref[0, 0, 0]
    t_ref[...] = (hx / (hy + EPS))[None]
    xt_ref[...] = x_ref[0]


def _rank_body(ar_ref, ac_ref, p_ref):
    i = pl.program_id(0)
    kr = _monotone_key(ar_ref[0])
    kc = _monotone_key(ac_ref[0])
    ii = lax.broadcasted_iota(jnp.int32, (1, S), 1)
    acc = jnp.zeros((1, S), jnp.float32)
    CH = 512
    ones = jnp.ones((1, CH), jnp.float32)
    for jc in range(S // CH):
        kj = kc[jc * CH:(jc + 1) * CH]
        jj = lax.broadcasted_iota(jnp.int32, (CH, 1), 0) + jc * CH
        hit = (kj < kr) | ((kj == kr) & (jj < ii))
        acc = acc + lax.dot_general(
            ones, jnp.where(hit, 1.0, 0.0), (((1,), (0,)), ((), ())),
            precision=lax.Precision.DEFAULT,
            preferred_element_type=jnp.float32)
    p_row = acc + jnp.float32(1.0) * (i * S)
    p_ref[...] = p_row.astype(jnp.int32).reshape(S)


def _conv_body(xs_ref, w_ref, bc_ref, y_ref):
    xsb = xs_ref[...]
    dn = jnp.concatenate([xsb[S - 1:], xsb[:S - 1]], axis=0)
    up = jnp.concatenate([xsb[1:], xsb[:1]], axis=0)
    w = w_ref[...]
    dims = (((1,), (1,)), ((), ()))
    y = lax.dot_general(dn, w[:, :, 0], dims,
                        precision=lax.Precision.DEFAULT,
                        preferred_element_type=jnp.float32)
    y += lax.dot_general(xsb, w[:, :, 1], dims,
                         precision=lax.Precision.DEFAULT,
                         preferred_element_type=jnp.float32)
    y += lax.dot_general(up, w[:, :, 2], dims,
                         precision=lax.Precision.DEFAULT,
                         preferred_element_type=jnp.float32)
    y_ref[...] = y + bc_ref[0]


def _sc_permute_body(xt_hbm, p_hbm, xs_hbm, buf, idxv, sem):
    wid = lax.axis_index("s") * 2 + lax.axis_index("c")
    for k in range(CPW):
        start = (wid * CPW + k) * CHUNK
        pltpu.sync_copy(xt_hbm.at[pl.ds(start, CHUNK)], buf)
        pltpu.sync_copy(p_hbm.at[pl.ds(start, CHUNK)], idxv)
        pltpu.async_copy(buf, xs_hbm.at[idxv], sem).wait()


def _sc_unpermute_body(y_hbm, p_hbm, out_hbm, buf, idxv, sem):
    wid = lax.axis_index("s") * 2 + lax.axis_index("c")
    for k in range(CPW):
        chunk = wid * CPW + k
        start = chunk * CHUNK
        bh = chunk // (S // CHUNK)
        s0 = (chunk % (S // CHUNK)) * CHUNK
        b = bh // H
        h = bh % H
        pltpu.sync_copy(p_hbm.at[pl.ds(start, CHUNK)], idxv)
        pltpu.async_copy(y_hbm.at[idxv], buf, sem).wait()
        pltpu.sync_copy(buf, out_hbm.at[pl.ds(b * S + s0, CHUNK), h])


def kernel(x, W_hash, b_hash, W_conv, b_conv):
    hh = jnp.arange(H)
    wnum = W_hash[hh // 2, :, hh % 2].reshape(H, 1, DH)
    wden = W_hash[H // 2 + hh // 2, :, hh % 2].reshape(H, 1, DH)
    bnum = b_hash[hh // 2, hh % 2].reshape(H, 1, 1)
    bden = b_hash[H // 2 + hh // 2, hh % 2].reshape(H, 1, 1)
    b_conv3 = b_conv.reshape(H, 1, DH)

    xt, t3 = pl.pallas_call(
        _hash_body,
        grid=(BH,),
        in_specs=[
            pl.BlockSpec((1, S, DH), lambda i: (i // H, 0, i % H)),
            pl.BlockSpec((1, S, DH), lambda i: (i // H, 0, (i % H) // 2)),
            pl.BlockSpec((1, S, DH),
                         lambda i: (i // H, 0, H // 2 + (i % H) // 2)),
            pl.BlockSpec((1, 1, DH), lambda i: (i % H, 0, 0)),
            pl.BlockSpec((1, 1, DH), lambda i: (i % H, 0, 0)),
            pl.BlockSpec((1, 1, 1), lambda i: (i % H, 0, 0)),
            pl.BlockSpec((1, 1, 1), lambda i: (i % H, 0, 0)),
        ],
        out_specs=[
            pl.BlockSpec((S, DH), lambda i: (i, 0)),
            pl.BlockSpec((1, 1, S), lambda i: (i, 0, 0)),
        ],
        out_shape=[
            jax.ShapeDtypeStruct((R, DH), jnp.float32),
            jax.ShapeDtypeStruct((BH, 1, S), jnp.float32),
        ],
    )(x, x, x, wnum, wden, bnum, bden)

    angles3 = jnp.arctan(t3)
    angles_col = jnp.swapaxes(angles3, 1, 2)

    P = pl.pallas_call(
        _rank_body,
        grid=(BH,),
        in_specs=[
            pl.BlockSpec((1, 1, S), lambda i: (i, 0, 0)),
            pl.BlockSpec((1, S, 1), lambda i: (i, 0, 0)),
        ],
        out_specs=pl.BlockSpec((S,), lambda i: (i,)),
        out_shape=jax.ShapeDtypeStruct((R,), jnp.int32),
    )(angles3, angles_col)

    mesh = plsc.VectorSubcoreMesh(core_axis_name="c", subcore_axis_name="s")

    sc_permute = functools.partial(
        pl.kernel,
        mesh=mesh,
        out_type=jax.ShapeDtypeStruct((R, DH), jnp.float32),
        scratch_types=[
            pltpu.VMEM((CHUNK, DH), jnp.float32),
            pltpu.VMEM((CHUNK,), jnp.int32),
            pltpu.SemaphoreType.DMA,
        ],
    )(_sc_permute_body)
    xs = sc_permute(xt, P)

    y = pl.pallas_call(
        _conv_body,
        grid=(BH,),
        in_specs=[
            pl.BlockSpec((S, DH), lambda i: (i, 0)),
            pl.BlockSpec((DH, DH, K), lambda i: (i % H, 0, 0)),
            pl.BlockSpec((1, 1, DH), lambda i: (i % H, 0, 0)),
        ],
        out_specs=pl.BlockSpec((S, DH), lambda i: (i, 0)),
        out_shape=jax.ShapeDtypeStruct((R, DH), jnp.float32),
    )(xs, W_conv, b_conv3)

    sc_unpermute = functools.partial(
        pl.kernel,
        mesh=mesh,
        out_type=jax.ShapeDtypeStruct((B * S, H, DH), jnp.float32),
        scratch_types=[
            pltpu.VMEM((CHUNK, DH), jnp.float32),
            pltpu.VMEM((CHUNK,), jnp.int32),
            pltpu.SemaphoreType.DMA,
        ],
    )(_sc_unpermute_body)
    out3 = sc_unpermute(y, P)

    return out3.reshape(B, S, D)


# speedup vs baseline: 3.5507x; 1.4995x over previous
import functools

import jax
import jax.numpy as jnp
from jax import lax
from jax.experimental import pallas as pl
from jax.experimental.pallas import tpu as pltpu
from jax.experimental.pallas import tpu_sc as plsc

B, S, D, H = 2, 2048, 4096, 32
DH = D // H
K = 3
EPS = 1e-4
BH = B * H
R = B * H * S
CHUNK = 512
NW = 32
CPW = R // (CHUNK * NW)


def _monotone_key(v):
    b = lax.bitcast_convert_type(v, jnp.int32)
    m = lax.shift_right_arithmetic(b, 31)
    return b ^ (m & jnp.int32(0x7FFFFFFF))


def _hash_body(x_ref, xa_ref, xb2_ref, wn_ref, wd_ref, bn_ref,
               bd_ref, xt_ref, t_ref):
    dims = (((1,), (1,)), ((), ()))
    hx = lax.dot_general(
        wn_ref[0], xa_ref[0], dims,
        precision=lax.Precision.DEFAULT,
        preferred_element_type=jnp.float32) + bn_ref[0, 0, 0]
    hy = lax.dot_general(
        wd_ref[0], xb2_ref[0], dims,
        precision=lax.Precision.DEFAULT,
        preferred_element_type=jnp.float32) + bd_ref[0, 0, 0]
    t_ref[...] = (hx / (hy + EPS))[None]
    xt_ref[...] = x_ref[0]


def _rank_body(ar_ref, ac_ref, p_ref):
    i = pl.program_id(0)
    kr = _monotone_key(ar_ref[0])
    kc = _monotone_key(ac_ref[0])
    ii = lax.broadcasted_iota(jnp.int32, (1, S), 1)
    acc = jnp.zeros((1, S), jnp.float32)
    CH = 512
    ones = jnp.ones((1, CH), jnp.float32)
    for jc in range(S // CH):
        kj = kc[jc * CH:(jc + 1) * CH]
        jj = lax.broadcasted_iota(jnp.int32, (CH, 1), 0) + jc * CH
        hit = (kj < kr) | ((kj == kr) & (jj < ii))
        acc = acc + lax.dot_general(
            ones, jnp.where(hit, 1.0, 0.0), (((1,), (0,)), ((), ())),
            precision=lax.Precision.DEFAULT,
            preferred_element_type=jnp.float32)
    p_row = acc + jnp.float32(1.0) * (i * S)
    p_ref[...] = p_row.astype(jnp.int32).reshape(S)


def _conv_body(xs_ref, w_ref, bc_ref, y_ref):
    xsb = xs_ref[...]
    dn = pltpu.roll(xsb, 1, 0)
    up = pltpu.roll(xsb, S - 1, 0)
    w = w_ref[0]
    dims = (((1,), (0,)), ((), ()))
    y = lax.dot_general(dn, w[0], dims,
                        precision=lax.Precision.DEFAULT,
                        preferred_element_type=jnp.float32)
    y += lax.dot_general(xsb, w[1], dims,
                         precision=lax.Precision.DEFAULT,
                         preferred_element_type=jnp.float32)
    y += lax.dot_general(up, w[2], dims,
                         precision=lax.Precision.DEFAULT,
                         preferred_element_type=jnp.float32)
    y_ref[...] = y + bc_ref[0]


def _sc_permute_body(xt_hbm, p_hbm, xs_hbm, buf, idxv, sem):
    wid = lax.axis_index("s") * 2 + lax.axis_index("c")
    for k in range(CPW):
        start = (wid * CPW + k) * CHUNK
        pltpu.sync_copy(xt_hbm.at[pl.ds(start, CHUNK)], buf)
        pltpu.sync_copy(p_hbm.at[pl.ds(start, CHUNK)], idxv)
        pltpu.async_copy(buf, xs_hbm.at[idxv], sem).wait()


def _sc_unpermute_body(y_hbm, p_hbm, out_hbm, buf, idxv, sem):
    wid = lax.axis_index("s") * 2 + lax.axis_index("c")
    for k in range(CPW):
        chunk = wid * CPW + k
        start = chunk * CHUNK
        bh = chunk // (S // CHUNK)
        s0 = (chunk % (S // CHUNK)) * CHUNK
        b = bh // H
        h = bh % H
        pltpu.sync_copy(p_hbm.at[pl.ds(start, CHUNK)], idxv)
        pltpu.async_copy(y_hbm.at[idxv], buf, sem).wait()
        pltpu.sync_copy(buf, out_hbm.at[pl.ds(b * S + s0, CHUNK), h])


def kernel(x, W_hash, b_hash, W_conv, b_conv):
    hh = jnp.arange(H)
    wnum = W_hash[hh // 2, :, hh % 2].reshape(H, 1, DH)
    wden = W_hash[H // 2 + hh // 2, :, hh % 2].reshape(H, 1, DH)
    bnum = b_hash[hh // 2, hh % 2].reshape(H, 1, 1)
    bden = b_hash[H // 2 + hh // 2, hh % 2].reshape(H, 1, 1)
    b_conv3 = b_conv.reshape(H, 1, DH)

    xt, t3 = pl.pallas_call(
        _hash_body,
        grid=(BH,),
        in_specs=[
            pl.BlockSpec((1, S, DH), lambda i: (i // H, 0, i % H)),
            pl.BlockSpec((1, S, DH), lambda i: (i // H, 0, (i % H) // 2)),
            pl.BlockSpec((1, S, DH),
                         lambda i: (i // H, 0, H // 2 + (i % H) // 2)),
            pl.BlockSpec((1, 1, DH), lambda i: (i % H, 0, 0)),
            pl.BlockSpec((1, 1, DH), lambda i: (i % H, 0, 0)),
            pl.BlockSpec((1, 1, 1), lambda i: (i % H, 0, 0)),
            pl.BlockSpec((1, 1, 1), lambda i: (i % H, 0, 0)),
        ],
        out_specs=[
            pl.BlockSpec((S, DH), lambda i: (i, 0)),
            pl.BlockSpec((1, 1, S), lambda i: (i, 0, 0)),
        ],
        out_shape=[
            jax.ShapeDtypeStruct((R, DH), jnp.float32),
            jax.ShapeDtypeStruct((BH, 1, S), jnp.float32),
        ],
    )(x, x, x, wnum, wden, bnum, bden)

    angles3 = jnp.arctan(t3)
    angles_col = jnp.swapaxes(angles3, 1, 2)

    P = pl.pallas_call(
        _rank_body,
        grid=(BH,),
        in_specs=[
            pl.BlockSpec((1, 1, S), lambda i: (i, 0, 0)),
            pl.BlockSpec((1, S, 1), lambda i: (i, 0, 0)),
        ],
        out_specs=pl.BlockSpec((S,), lambda i: (i,)),
        out_shape=jax.ShapeDtypeStruct((R,), jnp.int32),
    )(angles3, angles_col)

    mesh = plsc.VectorSubcoreMesh(core_axis_name="c", subcore_axis_name="s")

    sc_permute = functools.partial(
        pl.kernel,
        mesh=mesh,
        out_type=jax.ShapeDtypeStruct((R, DH), jnp.float32),
        scratch_types=[
            pltpu.VMEM((CHUNK, DH), jnp.float32),
            pltpu.VMEM((CHUNK,), jnp.int32),
            pltpu.SemaphoreType.DMA,
        ],
    )(_sc_permute_body)
    xs = sc_permute(xt, P)

    wk4 = jnp.transpose(W_conv.reshape(H, DH, DH, K), (0, 3, 2, 1))
    y = pl.pallas_call(
        _conv_body,
        grid=(BH,),
        in_specs=[
            pl.BlockSpec((S, DH), lambda i: (i, 0)),
            pl.BlockSpec((1, K, DH, DH), lambda i: (i % H, 0, 0, 0)),
            pl.BlockSpec((1, 1, DH), lambda i: (i % H, 0, 0)),
        ],
        out_specs=pl.BlockSpec((S, DH), lambda i: (i, 0)),
        out_shape=jax.ShapeDtypeStruct((R, DH), jnp.float32),
    )(xs, wk4, b_conv3)

    sc_unpermute = functools.partial(
        pl.kernel,
        mesh=mesh,
        out_type=jax.ShapeDtypeStruct((B * S, H, DH), jnp.float32),
        scratch_types=[
            pltpu.VMEM((CHUNK, DH), jnp.float32),
            pltpu.VMEM((CHUNK,), jnp.int32),
            pltpu.SemaphoreType.DMA,
        ],
    )(_sc_unpermute_body)
    out3 = sc_unpermute(y, P)

    return out3.reshape(B, S, D)


# speedup vs baseline: 3.5769x; 1.0074x over previous
import functools

import jax
import jax.numpy as jnp
from jax import lax
from jax.experimental import pallas as pl
from jax.experimental.pallas import tpu as pltpu
from jax.experimental.pallas import tpu_sc as plsc

B, S, D, H = 2, 2048, 4096, 32
DH = D // H
K = 3
EPS = 1e-4
BH = B * H
R = B * H * S
CHUNK = 256
NW = 32
CPW = R // (CHUNK * NW)


def _monotone_key(v):
    b = lax.bitcast_convert_type(v, jnp.int32)
    m = lax.shift_right_arithmetic(b, 31)
    return b ^ (m & jnp.int32(0x7FFFFFFF))


def _hash_body(x_ref, xa_ref, xb2_ref, wn_ref, wd_ref, bn_ref,
               bd_ref, xt_ref, t_ref):
    dims = (((1,), (1,)), ((), ()))
    hx = lax.dot_general(
        wn_ref[0], xa_ref[0], dims,
        precision=lax.Precision.DEFAULT,
        preferred_element_type=jnp.float32) + bn_ref[0, 0, 0]
    hy = lax.dot_general(
        wd_ref[0], xb2_ref[0], dims,
        precision=lax.Precision.DEFAULT,
        preferred_element_type=jnp.float32) + bd_ref[0, 0, 0]
    t_ref[...] = (hx / (hy + EPS))[None]
    xt_ref[...] = x_ref[0]


def _rank_body(ar_ref, ac_ref, p_ref):
    i = pl.program_id(0)
    kr = _monotone_key(ar_ref[0])
    kc = _monotone_key(ac_ref[0])
    ii = lax.broadcasted_iota(jnp.int32, (1, S), 1)
    acc = jnp.zeros((1, S), jnp.float32)
    CH = 512
    ones = jnp.ones((1, CH), jnp.float32)
    for jc in range(S // CH):
        kj = kc[jc * CH:(jc + 1) * CH]
        jj = lax.broadcasted_iota(jnp.int32, (CH, 1), 0) + jc * CH
        hit = (kj < kr) | ((kj == kr) & (jj < ii))
        acc = acc + lax.dot_general(
            ones, jnp.where(hit, 1.0, 0.0), (((1,), (0,)), ((), ())),
            precision=lax.Precision.DEFAULT,
            preferred_element_type=jnp.float32)
    p_row = acc + jnp.float32(1.0) * (i * S)
    p_ref[...] = p_row.astype(jnp.int32).reshape(S)


def _conv_body(xs_ref, w_ref, bc_ref, y_ref):
    xsb = xs_ref[...]
    dn = pltpu.roll(xsb, 1, 0)
    up = pltpu.roll(xsb, S - 1, 0)
    w = w_ref[0]
    dims = (((1,), (0,)), ((), ()))
    y = lax.dot_general(dn, w[0], dims,
                        precision=lax.Precision.DEFAULT,
                        preferred_element_type=jnp.float32)
    y += lax.dot_general(xsb, w[1], dims,
                         precision=lax.Precision.DEFAULT,
                         preferred_element_type=jnp.float32)
    y += lax.dot_general(up, w[2], dims,
                         precision=lax.Precision.DEFAULT,
                         preferred_element_type=jnp.float32)
    y_ref[...] = y + bc_ref[0]


def _sc_permute_body(xt_hbm, p_hbm, xs_hbm, buf0, buf1, idx0, idx1,
                     lsem0, lsem1, wsem0, wsem1):
    wid = lax.axis_index("s") * 2 + lax.axis_index("c")
    bufs = (buf0, buf1)
    idxs = (idx0, idx1)
    lsems = (lsem0, lsem1)
    wsems = (wsem0, wsem1)

    def issue_loads(k):
        b = k % 2
        start = (wid * CPW + k) * CHUNK
        c1 = pltpu.make_async_copy(xt_hbm.at[pl.ds(start, CHUNK)],
                                   bufs[b], lsems[b])
        c2 = pltpu.make_async_copy(p_hbm.at[pl.ds(start, CHUNK)],
                                   idxs[b], lsems[b])
        c1.start()
        c2.start()
        return (c1, c2)

    loads = {0: issue_loads(0)}
    writes = {}
    for k in range(CPW):
        b = k % 2
        if k >= 1:
            writes[k - 1].wait()
        if k + 1 < CPW:
            loads[k + 1] = issue_loads(k + 1)
        for h in loads[k]:
            h.wait()
        writes[k] = pltpu.make_async_copy(bufs[b], xs_hbm.at[idxs[b]],
                                          wsems[b])
        writes[k].start()
    writes[CPW - 1].wait()


def _sc_unpermute_body(y_hbm, p_hbm, out_hbm, buf0, buf1, idx0, idx1,
                       lsem0, lsem1, gsem0, gsem1, wsem0, wsem1):
    wid = lax.axis_index("s") * 2 + lax.axis_index("c")
    bufs = (buf0, buf1)
    idxs = (idx0, idx1)
    lsems = (lsem0, lsem1)
    gsems = (gsem0, gsem1)
    wsems = (wsem0, wsem1)

    def issue_idx(k):
        b = k % 2
        start = (wid * CPW + k) * CHUNK
        c = pltpu.make_async_copy(p_hbm.at[pl.ds(start, CHUNK)],
                                  idxs[b], lsems[b])
        c.start()
        return c

    def issue_gather(k):
        b = k % 2
        c = pltpu.make_async_copy(y_hbm.at[idxs[b]], bufs[b], gsems[b])
        c.start()
        return c

    def issue_write(k):
        b = k % 2
        chunk = wid * CPW + k
        bh = chunk // (S // CHUNK)
        s0 = (chunk % (S // CHUNK)) * CHUNK
        bb = bh // H
        h = bh % H
        c = pltpu.make_async_copy(
            bufs[b], out_hbm.at[pl.ds(bb * S + s0, CHUNK), h],
            wsems[b])
        c.start()
        return c

    idxl = {0: issue_idx(0)}
    gaths = {}
    writes = {}
    for k in range(CPW):
        b = k % 2
        if k + 1 < CPW:
            idxl[k + 1] = issue_idx(k + 1)
        if k >= 1:
            writes[k - 1].wait()
        idxl[k].wait()
        gaths[k] = issue_gather(k)
        gaths[k].wait()
        writes[k] = issue_write(k)
    writes[CPW - 1].wait()


def kernel(x, W_hash, b_hash, W_conv, b_conv):
    hh = jnp.arange(H)
    wnum = W_hash[hh // 2, :, hh % 2].reshape(H, 1, DH)
    wden = W_hash[H // 2 + hh // 2, :, hh % 2].reshape(H, 1, DH)
    bnum = b_hash[hh // 2, hh % 2].reshape(H, 1, 1)
    bden = b_hash[H // 2 + hh // 2, hh % 2].reshape(H, 1, 1)
    b_conv3 = b_conv.reshape(H, 1, DH)

    xt, t3 = pl.pallas_call(
        _hash_body,
        grid=(BH,),
        in_specs=[
            pl.BlockSpec((1, S, DH), lambda i: (i // H, 0, i % H)),
            pl.BlockSpec((1, S, DH), lambda i: (i // H, 0, (i % H) // 2)),
            pl.BlockSpec((1, S, DH),
                         lambda i: (i // H, 0, H // 2 + (i % H) // 2)),
            pl.BlockSpec((1, 1, DH), lambda i: (i % H, 0, 0)),
            pl.BlockSpec((1, 1, DH), lambda i: (i % H, 0, 0)),
            pl.BlockSpec((1, 1, 1), lambda i: (i % H, 0, 0)),
            pl.BlockSpec((1, 1, 1), lambda i: (i % H, 0, 0)),
        ],
        out_specs=[
            pl.BlockSpec((S, DH), lambda i: (i, 0)),
            pl.BlockSpec((1, 1, S), lambda i: (i, 0, 0)),
        ],
        out_shape=[
            jax.ShapeDtypeStruct((R, DH), jnp.float32),
            jax.ShapeDtypeStruct((BH, 1, S), jnp.float32),
        ],
    )(x, x, x, wnum, wden, bnum, bden)

    angles3 = jnp.arctan(t3)
    angles_col = jnp.swapaxes(angles3, 1, 2)

    P = pl.pallas_call(
        _rank_body,
        grid=(BH,),
        in_specs=[
            pl.BlockSpec((1, 1, S), lambda i: (i, 0, 0)),
            pl.BlockSpec((1, S, 1), lambda i: (i, 0, 0)),
        ],
        out_specs=pl.BlockSpec((S,), lambda i: (i,)),
        out_shape=jax.ShapeDtypeStruct((R,), jnp.int32),
    )(angles3, angles_col)

    mesh = plsc.VectorSubcoreMesh(core_axis_name="c", subcore_axis_name="s")

    sc_permute = functools.partial(
        pl.kernel,
        mesh=mesh,
        out_type=jax.ShapeDtypeStruct((R, DH), jnp.float32),
        scratch_types=[
            pltpu.VMEM((CHUNK, DH), jnp.float32),
            pltpu.VMEM((CHUNK, DH), jnp.float32),
            pltpu.VMEM((CHUNK,), jnp.int32),
            pltpu.VMEM((CHUNK,), jnp.int32),
            pltpu.SemaphoreType.DMA,
            pltpu.SemaphoreType.DMA,
            pltpu.SemaphoreType.DMA,
            pltpu.SemaphoreType.DMA,
        ],
    )(_sc_permute_body)
    xs = sc_permute(xt, P)

    wk4 = jnp.transpose(W_conv.reshape(H, DH, DH, K), (0, 3, 2, 1))
    y = pl.pallas_call(
        _conv_body,
        grid=(BH,),
        in_specs=[
            pl.BlockSpec((S, DH), lambda i: (i, 0)),
            pl.BlockSpec((1, K, DH, DH), lambda i: (i % H, 0, 0, 0)),
            pl.BlockSpec((1, 1, DH), lambda i: (i % H, 0, 0)),
        ],
        out_specs=pl.BlockSpec((S, DH), lambda i: (i, 0)),
        out_shape=jax.ShapeDtypeStruct((R, DH), jnp.float32),
    )(xs, wk4, b_conv3)

    sc_unpermute = functools.partial(
        pl.kernel,
        mesh=mesh,
        out_type=jax.ShapeDtypeStruct((B * S, H, DH), jnp.float32),
        scratch_types=[
            pltpu.VMEM((CHUNK, DH), jnp.float32),
            pltpu.VMEM((CHUNK, DH), jnp.float32),
            pltpu.VMEM((CHUNK,), jnp.int32),
            pltpu.VMEM((CHUNK,), jnp.int32),
            pltpu.SemaphoreType.DMA,
            pltpu.SemaphoreType.DMA,
            pltpu.SemaphoreType.DMA,
            pltpu.SemaphoreType.DMA,
            pltpu.SemaphoreType.DMA,
            pltpu.SemaphoreType.DMA,
        ],
    )(_sc_unpermute_body)
    out3 = sc_unpermute(y, P)

    return out3.reshape(B, S, D)
